# speedup vs baseline: 13.2449x; 13.2449x over previous
import functools

import jax
import jax.numpy as jnp
from jax import lax
from jax.experimental import pallas as pl
from jax.experimental.pallas import tpu as pltpu
from jax.experimental.pallas import tpu_sc as plsc

KNN = 3
NQ = 8192
NC = 2048
DC = 256
DS = 128
DH = 256

TQ = 256


def _knn_body(ps_ref, bs_ref, pt_ref, bt_ref, idx_ref, wn_ref):
    d2 = jnp.zeros((TQ, NC), jnp.float32)
    for c in range(3):
        diff = ps_ref[:, c:c + 1] - pt_ref[c:c + 1, :]
        d2 = d2 + diff * diff
    mask = bs_ref[:, :] != bt_ref[:, :]
    d2 = jnp.where(mask, jnp.float32(1e30), d2)
    cols = lax.broadcasted_iota(jnp.int32, (TQ, NC), 1)
    vals, idxs = [], []
    for _ in range(KNN):
        v = jnp.min(d2, axis=1, keepdims=True)
        i = jnp.min(jnp.where(d2 == v, cols, NC), axis=1, keepdims=True)
        vals.append(v)
        idxs.append(i)
        d2 = jnp.where(cols == i, jnp.float32(1e38), d2)
    w = [1.0 / jnp.maximum(v, jnp.float32(1e-16)) for v in vals]
    den = w[0] + w[1] + w[2]
    wn = [wi / den for wi in w]
    idx_ref[...] = jnp.concatenate(
        idxs + [jnp.zeros((TQ, 8 - KNN), jnp.int32)], axis=1)
    wn_ref[...] = jnp.concatenate(
        wn + [jnp.zeros((TQ, 8 - KNN), jnp.float32)], axis=1)


def _knn_call(pos_skip, bs_col, pos_t, batch_row):
    return pl.pallas_call(
        _knn_body,
        grid=(NQ // TQ,),
        in_specs=[
            pl.BlockSpec((TQ, 3), lambda i: (i, 0)),
            pl.BlockSpec((TQ, 1), lambda i: (i, 0)),
            pl.BlockSpec((3, NC), lambda i: (0, 0)),
            pl.BlockSpec((1, NC), lambda i: (0, 0)),
        ],
        out_specs=[
            pl.BlockSpec((TQ, 8), lambda i: (i, 0)),
            pl.BlockSpec((TQ, 8), lambda i: (i, 0)),
        ],
        out_shape=[
            jax.ShapeDtypeStruct((NQ, 8), jnp.int32),
            jax.ShapeDtypeStruct((NQ, 8), jnp.float32),
        ],
    )(pos_skip, bs_col, pos_t, batch_row)


NROWS = KNN * NQ
NW = 32
RPW = NROWS // NW
CH = 128
NCHUNK = RPW // CH


def _gather_body(x_hbm, idx_hbm, out_hbm, idx_v, rows_v, sem):
    wid = lax.axis_index("s") * 2 + lax.axis_index("c")
    base = pl.multiple_of(wid * RPW, CH)
    for c in range(NCHUNK):
        off = pl.multiple_of(base + c * CH, CH)
        pltpu.sync_copy(idx_hbm.at[pl.ds(off, CH)], idx_v)
        pltpu.async_copy(x_hbm.at[idx_v], rows_v, sem).wait()
        pltpu.sync_copy(rows_v, out_hbm.at[pl.ds(off, CH)])


def _gather_call(x, idx_flat):
    mesh = plsc.VectorSubcoreMesh(core_axis_name="c", subcore_axis_name="s")
    fn = functools.partial(
        pl.kernel,
        mesh=mesh,
        out_type=jax.ShapeDtypeStruct((NROWS, DC), jnp.float32),
        scratch_types=[
            pltpu.VMEM((CH,), jnp.int32),
            pltpu.VMEM((CH, DC), jnp.float32),
            pltpu.SemaphoreType.DMA,
        ],
    )(_gather_body)
    return fn(x, idx_flat)


def _mlp_body(f_ref, wn_ref, xs_ref, w1a_ref, w1b_ref, b1_ref, w2_ref,
              b2_ref, out_ref):
    y = (wn_ref[:, 0:1] * f_ref[0]
         + wn_ref[:, 1:2] * f_ref[1]
         + wn_ref[:, 2:3] * f_ref[2])
    h = jnp.dot(y, w1a_ref[...], preferred_element_type=jnp.float32)
    h = h + jnp.dot(xs_ref[...], w1b_ref[...], preferred_element_type=jnp.float32)
    h = jnp.maximum(h + b1_ref[...], 0.0)
    out_ref[...] = (jnp.dot(h, w2_ref[...], preferred_element_type=jnp.float32)
                    + b2_ref[...])


def _mlp_call(feats3, wn8, x_skip, W1a, W1b, b1r, W2, b2r):
    return pl.pallas_call(
        _mlp_body,
        grid=(NQ // TQ,),
        in_specs=[
            pl.BlockSpec((KNN, TQ, DC), lambda i: (0, i, 0)),
            pl.BlockSpec((TQ, 8), lambda i: (i, 0)),
            pl.BlockSpec((TQ, DS), lambda i: (i, 0)),
            pl.BlockSpec((DC, DH), lambda i: (0, 0)),
            pl.BlockSpec((DS, DH), lambda i: (0, 0)),
            pl.BlockSpec((1, DH), lambda i: (0, 0)),
            pl.BlockSpec((DH, DH), lambda i: (0, 0)),
            pl.BlockSpec((1, DH), lambda i: (0, 0)),
        ],
        out_specs=pl.BlockSpec((TQ, DH), lambda i: (i, 0)),
        out_shape=jax.ShapeDtypeStruct((NQ, DH), jnp.float32),
    )(feats3, wn8, x_skip, W1a, W1b, b1r, W2, b2r)


def kernel(x, pos, batch, x_skip, pos_skip, batch_skip, W1, b1, W2, b2):
    pos_t = pos.T
    batch_row = batch.reshape(1, NC)
    bs_col = batch_skip.reshape(NQ, 1)
    idx8, wn8 = _knn_call(pos_skip, bs_col, pos_t, batch_row)
    idx_flat = idx8[:, :KNN].T.reshape(-1)
    feats = _gather_call(x, idx_flat)
    feats3 = feats.reshape(KNN, NQ, DC)
    out = _mlp_call(feats3, wn8, x_skip, W1[:DC], W1[DC:],
                    b1.reshape(1, DH), W2, b2.reshape(1, DH))
    return (out, pos_skip, batch_skip)
